# baseline (speedup 1.0000x reference)
import jax
import jax.numpy as jnp
from jax import lax
from jax.experimental import pallas as pl
from jax.experimental.pallas import tpu as pltpu

_E = 8
_K = 2
_Z_LOSS_COEF = 0.01
_AUX_LOSS_COEF = 0.01
_B = 256
_HB = 512


def _router_kernel(x_ref, wr_ref, idx_ref, w_ref, aux_ref, xbf_ref):
    x = x_ref[...]
    logits = jnp.dot(x, wr_ref[...], preferred_element_type=jnp.float32)
    mx = jnp.max(logits, axis=-1, keepdims=True)
    ex = jnp.exp(logits - mx)
    den = jnp.sum(ex, axis=-1, keepdims=True)
    probs = ex / den

    T, E = logits.shape
    iota = lax.broadcasted_iota(jnp.int32, (T, E), 1)
    m1 = jnp.max(probs, axis=-1, keepdims=True)
    idx1 = jnp.min(jnp.where(probs == m1, iota, E), axis=-1, keepdims=True)
    sel1 = iota == idx1
    pm = jnp.where(sel1, -1.0, probs)
    m2 = jnp.max(pm, axis=-1, keepdims=True)
    idx2 = jnp.min(jnp.where(pm == m2, iota, E), axis=-1, keepdims=True)
    sel2 = iota == idx2
    s = m1 + m2
    idx_ref[...] = jnp.concatenate([idx1, idx2], axis=1)
    w_ref[...] = jnp.concatenate([m1 / s, m2 / s], axis=1)

    usage = jnp.mean(probs, axis=0, keepdims=True)
    selection = (
        jnp.mean(sel1.astype(jnp.float32) + sel2.astype(jnp.float32), axis=0,
                 keepdims=True) / _K)
    lb = E * jnp.sum(usage * selection)
    lse = jnp.log(den) + mx
    z = jnp.mean(lse * lse)
    aux_ref[...] = jnp.reshape(_AUX_LOSS_COEF * lb + _Z_LOSS_COEF * z, (1, 1))

    xbf_ref[...] = x.astype(jnp.bfloat16)


def _h_kernel(te_ref, tv_ref, xbf_ref, w1_ref, b1_ref, stok_ref, sw_ref,
              h_ref, xs_ref, w1bf_ref):
    h = pl.program_id(0)
    t = pl.program_id(1)
    T = xbf_ref.shape[0]
    valid = tv_ref[t] > 0
    new_w = (t == 0) | (te_ref[t] != te_ref[jnp.maximum(t - 1, 0)])

    @pl.when(valid & (h == 0))
    def _gather():
        stok = stok_ref[0]
        iota = lax.broadcasted_iota(jnp.int32, (_B, T), 1)
        oh = (iota == stok).astype(jnp.bfloat16)
        xs_ref[pl.ds(t * _B, _B), :] = jnp.dot(
            oh, xbf_ref[...],
            preferred_element_type=jnp.float32).astype(jnp.bfloat16)

    @pl.when(valid & new_w)
    def _cast():
        w1bf_ref[...] = w1_ref[0].astype(jnp.bfloat16)

    @pl.when(valid)
    def _mlp1():
        xs = xs_ref[pl.ds(t * _B, _B), :]
        hpre = jnp.dot(xs, w1bf_ref[...],
                       preferred_element_type=jnp.float32) + b1_ref[0]
        hact = 0.5 * hpre * (1.0 + lax.erf(hpre * 0.7071067811865476))
        h_ref[...] = (hact * sw_ref[0]).astype(jnp.bfloat16)

    @pl.when(jnp.logical_not(valid))
    def _zero():
        h_ref[...] = jnp.zeros_like(h_ref)


def _y_kernel(te_ref, tv_ref, h_ref, w2_ref, b2_ref, stokr_ref, sw_ref,
              out_ref, w2bf_ref):
    t = pl.program_id(0)
    T = out_ref.shape[0]
    valid = tv_ref[t] > 0
    new_w = (t == 0) | (te_ref[t] != te_ref[jnp.maximum(t - 1, 0)])

    @pl.when(t == 0)
    def _init():
        out_ref[...] = jnp.zeros_like(out_ref)

    @pl.when(valid & new_w)
    def _cast():
        w2bf_ref[...] = w2_ref[0].astype(jnp.bfloat16)

    @pl.when(valid)
    def _compute():
        y = jnp.dot(h_ref[...], w2bf_ref[...],
                    preferred_element_type=jnp.float32)
        y = y + sw_ref[0] * b2_ref[0]
        stokr = stokr_ref[0]
        iota = lax.broadcasted_iota(jnp.int32, (T, _B), 0)
        ohT = (iota == stokr).astype(jnp.bfloat16)
        out_ref[...] += jnp.dot(ohT, y.astype(jnp.bfloat16),
                                preferred_element_type=jnp.float32)


def kernel(x, Wr, W1, b1, W2, b2):
    T, D = x.shape
    E = Wr.shape[1]
    H = W1.shape[2]
    P = T * _K
    ntiles = P // _B + _E
    P_pad = ntiles * _B

    idx, w, aux, xbf = pl.pallas_call(
        _router_kernel,
        out_shape=[
            jax.ShapeDtypeStruct((T, _K), jnp.int32),
            jax.ShapeDtypeStruct((T, _K), jnp.float32),
            jax.ShapeDtypeStruct((1, 1), jnp.float32),
            jax.ShapeDtypeStruct((T, D), jnp.bfloat16),
        ],
    )(x, Wr)

    idxf = idx.reshape(P)
    wf = w.reshape(P)
    oh = (idxf[:, None] == jnp.arange(E, dtype=jnp.int32)[None, :])
    oh = oh.astype(jnp.int32)
    csum = jnp.cumsum(oh, axis=0)
    ranks = jnp.sum(csum * oh, axis=1) - 1
    counts = csum[-1]
    ptiles = (counts + _B - 1) // _B
    group_start = (jnp.concatenate(
        [jnp.zeros(1, jnp.int32), jnp.cumsum(ptiles)])[:E] * _B)
    pos = group_start[idxf] + ranks
    slot_token = jnp.zeros((P_pad,), jnp.int32).at[pos].set(
        jnp.arange(P, dtype=jnp.int32) // _K)
    slot_w = jnp.zeros((P_pad,), jnp.float32).at[pos].set(wf)
    tile_ids = jnp.arange(ntiles, dtype=jnp.int32)
    tile_bound = jnp.cumsum(ptiles)
    tile_expert = jnp.minimum(
        jnp.searchsorted(tile_bound, tile_ids, side="right").astype(jnp.int32),
        E - 1)
    tile_valid = (tile_ids < tile_bound[-1]).astype(jnp.int32)

    stok_c = slot_token.reshape(ntiles, _B, 1)
    stok_r = slot_token.reshape(ntiles, 1, _B)
    sw_c = slot_w.reshape(ntiles, _B, 1)
    b1r = b1.reshape(E, 1, H)
    b2r = b2.reshape(E, 1, D)
    nh = H // _HB

    hmat = pl.pallas_call(
        _h_kernel,
        grid_spec=pltpu.PrefetchScalarGridSpec(
            num_scalar_prefetch=2,
            grid=(nh, ntiles),
            in_specs=[
                pl.BlockSpec((T, D), lambda h, t, te, tv: (0, 0)),
                pl.BlockSpec((1, D, _HB), lambda h, t, te, tv: (te[t], 0, h)),
                pl.BlockSpec((1, 1, _HB), lambda h, t, te, tv: (te[t], 0, h)),
                pl.BlockSpec((1, _B, 1), lambda h, t, te, tv: (t, 0, 0)),
                pl.BlockSpec((1, _B, 1), lambda h, t, te, tv: (t, 0, 0)),
            ],
            out_specs=pl.BlockSpec((_B, _HB), lambda h, t, te, tv: (t, h)),
            scratch_shapes=[
                pltpu.VMEM((P_pad, D), jnp.bfloat16),
                pltpu.VMEM((D, _HB), jnp.bfloat16),
            ],
        ),
        out_shape=jax.ShapeDtypeStruct((P_pad, H), jnp.bfloat16),
    )(tile_expert, tile_valid, xbf, W1, b1r, stok_c, sw_c)

    out = pl.pallas_call(
        _y_kernel,
        grid_spec=pltpu.PrefetchScalarGridSpec(
            num_scalar_prefetch=2,
            grid=(ntiles,),
            in_specs=[
                pl.BlockSpec((_B, H), lambda t, te, tv: (t, 0)),
                pl.BlockSpec((1, H, D), lambda t, te, tv: (te[t], 0, 0)),
                pl.BlockSpec((1, 1, D), lambda t, te, tv: (te[t], 0, 0)),
                pl.BlockSpec((1, 1, _B), lambda t, te, tv: (t, 0, 0)),
                pl.BlockSpec((1, _B, 1), lambda t, te, tv: (t, 0, 0)),
            ],
            out_specs=pl.BlockSpec((T, D), lambda t, te, tv: (0, 0)),
            scratch_shapes=[pltpu.VMEM((H, D), jnp.bfloat16)],
        ),
        out_shape=jax.ShapeDtypeStruct((T, D), jnp.float32),
    )(tile_expert, tile_valid, hmat, W2, b2r, stok_r, sw_c)

    return out, aux[0, 0]


# speedup vs baseline: 1.0067x; 1.0067x over previous
import jax
import jax.numpy as jnp
from jax import lax
from jax.experimental import pallas as pl
from jax.experimental.pallas import tpu as pltpu

_E = 8
_K = 2
_Z_LOSS_COEF = 0.01
_AUX_LOSS_COEF = 0.01
_B = 256
_HB = 512


def _router_kernel(x_ref, wr_ref, idx_ref, w_ref, aux_ref, xbf_ref):
    x = x_ref[...]
    logits = jnp.dot(x, wr_ref[...], preferred_element_type=jnp.float32)
    mx = jnp.max(logits, axis=-1, keepdims=True)
    ex = jnp.exp(logits - mx)
    den = jnp.sum(ex, axis=-1, keepdims=True)
    probs = ex / den

    T, E = logits.shape
    iota = lax.broadcasted_iota(jnp.int32, (T, E), 1)
    m1 = jnp.max(probs, axis=-1, keepdims=True)
    idx1 = jnp.min(jnp.where(probs == m1, iota, E), axis=-1, keepdims=True)
    sel1 = iota == idx1
    pm = jnp.where(sel1, -1.0, probs)
    m2 = jnp.max(pm, axis=-1, keepdims=True)
    idx2 = jnp.min(jnp.where(pm == m2, iota, E), axis=-1, keepdims=True)
    sel2 = iota == idx2
    s = m1 + m2
    idx_ref[...] = jnp.concatenate([idx1, idx2], axis=1)
    w_ref[...] = jnp.concatenate([m1 / s, m2 / s], axis=1)

    usage = jnp.mean(probs, axis=0, keepdims=True)
    selection = (
        jnp.mean(sel1.astype(jnp.float32) + sel2.astype(jnp.float32), axis=0,
                 keepdims=True) / _K)
    lb = E * jnp.sum(usage * selection)
    lse = jnp.log(den) + mx
    z = jnp.mean(lse * lse)
    aux_ref[...] = jnp.reshape(_AUX_LOSS_COEF * lb + _Z_LOSS_COEF * z, (1, 1))

    xbf_ref[...] = x.astype(jnp.bfloat16)


def _gather_kernel(tv_ref, xbf_ref, stok_ref, xs_ref):
    t = pl.program_id(0)
    T = xbf_ref.shape[0]
    valid = tv_ref[t] > 0

    @pl.when(valid)
    def _gather():
        stok = stok_ref[0]
        iota = lax.broadcasted_iota(jnp.int32, (_B, T), 1)
        oh = (iota == stok).astype(jnp.bfloat16)
        xs_ref[...] = jnp.dot(
            oh, xbf_ref[...],
            preferred_element_type=jnp.float32).astype(jnp.bfloat16)

    @pl.when(jnp.logical_not(valid))
    def _zero():
        xs_ref[...] = jnp.zeros_like(xs_ref)


def _mlp_kernel(te_ref, tv_ref, xs_ref, w1_ref, b1_ref, w2_ref, b2_ref,
                stokr_ref, sw_ref, out_ref, acc_ref, w1bf_ref, w2bf_ref):
    h = pl.program_id(0)
    t = pl.program_id(1)
    nh = pl.num_programs(0)
    T = out_ref.shape[0]
    valid = tv_ref[t] > 0
    new_w = (t == 0) | (te_ref[t] != te_ref[jnp.maximum(t - 1, 0)])

    @pl.when((h == 0) & (t == 0))
    def _init():
        out_ref[...] = jnp.zeros_like(out_ref)

    @pl.when(valid & new_w)
    def _cast():
        w1bf_ref[...] = w1_ref[0].astype(jnp.bfloat16)
        w2bf_ref[...] = w2_ref[0].astype(jnp.bfloat16)

    @pl.when(valid)
    def _mlp():
        xs = xs_ref[pl.ds(t * _B, _B), :]
        hpre = jnp.dot(xs, w1bf_ref[...],
                       preferred_element_type=jnp.float32) + b1_ref[0]
        hact = 0.5 * hpre * (1.0 + lax.erf(hpre * 0.7071067811865476))
        hblk = (hact * sw_ref[0]).astype(jnp.bfloat16)
        part = jnp.dot(hblk, w2bf_ref[...], preferred_element_type=jnp.float32)

        @pl.when(h == 0)
        def _set():
            acc_ref[pl.ds(t * _B, _B), :] = part

        @pl.when(h > 0)
        def _add():
            acc_ref[pl.ds(t * _B, _B), :] += part

    @pl.when(valid & (h == nh - 1))
    def _scatter():
        y = acc_ref[pl.ds(t * _B, _B), :] + sw_ref[0] * b2_ref[0]
        stokr = stokr_ref[0]
        iota = lax.broadcasted_iota(jnp.int32, (T, _B), 0)
        ohT = (iota == stokr).astype(jnp.bfloat16)
        out_ref[...] += jnp.dot(ohT, y.astype(jnp.bfloat16),
                                preferred_element_type=jnp.float32)


def kernel(x, Wr, W1, b1, W2, b2):
    T, D = x.shape
    E = Wr.shape[1]
    H = W1.shape[2]
    P = T * _K
    ntiles = P // _B + _E
    P_pad = ntiles * _B

    idx, w, aux, xbf = pl.pallas_call(
        _router_kernel,
        out_shape=[
            jax.ShapeDtypeStruct((T, _K), jnp.int32),
            jax.ShapeDtypeStruct((T, _K), jnp.float32),
            jax.ShapeDtypeStruct((1, 1), jnp.float32),
            jax.ShapeDtypeStruct((T, D), jnp.bfloat16),
        ],
    )(x, Wr)

    idxf = idx.reshape(P)
    wf = w.reshape(P)
    oh = (idxf[:, None] == jnp.arange(E, dtype=jnp.int32)[None, :])
    oh = oh.astype(jnp.int32)
    csum = jnp.cumsum(oh, axis=0)
    ranks = jnp.sum(csum * oh, axis=1) - 1
    counts = csum[-1]
    ptiles = (counts + _B - 1) // _B
    group_start = (jnp.concatenate(
        [jnp.zeros(1, jnp.int32), jnp.cumsum(ptiles)])[:E] * _B)
    pos = group_start[idxf] + ranks
    slot_token = jnp.zeros((P_pad,), jnp.int32).at[pos].set(
        jnp.arange(P, dtype=jnp.int32) // _K)
    slot_w = jnp.zeros((P_pad,), jnp.float32).at[pos].set(wf)
    tile_ids = jnp.arange(ntiles, dtype=jnp.int32)
    tile_bound = jnp.cumsum(ptiles)
    tile_expert = jnp.minimum(
        jnp.searchsorted(tile_bound, tile_ids, side="right").astype(jnp.int32),
        E - 1)
    tile_valid = (tile_ids < tile_bound[-1]).astype(jnp.int32)

    stok_c = slot_token.reshape(ntiles, _B, 1)
    stok_r = slot_token.reshape(ntiles, 1, _B)
    sw_c = slot_w.reshape(ntiles, _B, 1)
    b1r = b1.reshape(E, 1, H)
    b2r = b2.reshape(E, 1, D)
    nh = H // _HB

    xs = pl.pallas_call(
        _gather_kernel,
        grid_spec=pltpu.PrefetchScalarGridSpec(
            num_scalar_prefetch=1,
            grid=(ntiles,),
            in_specs=[
                pl.BlockSpec((T, D), lambda t, tv: (0, 0)),
                pl.BlockSpec((1, _B, 1), lambda t, tv: (t, 0, 0)),
            ],
            out_specs=pl.BlockSpec((_B, D), lambda t, tv: (t, 0)),
        ),
        out_shape=jax.ShapeDtypeStruct((P_pad, D), jnp.bfloat16),
    )(tile_valid, xbf, stok_c)

    out = pl.pallas_call(
        _mlp_kernel,
        grid_spec=pltpu.PrefetchScalarGridSpec(
            num_scalar_prefetch=2,
            grid=(nh, ntiles),
            in_specs=[
                pl.BlockSpec((P_pad, D), lambda h, t, te, tv: (0, 0)),
                pl.BlockSpec((1, D, _HB), lambda h, t, te, tv: (te[t], 0, h)),
                pl.BlockSpec((1, 1, _HB), lambda h, t, te, tv: (te[t], 0, h)),
                pl.BlockSpec((1, _HB, D), lambda h, t, te, tv: (te[t], h, 0)),
                pl.BlockSpec((1, 1, D), lambda h, t, te, tv: (te[t], 0, 0)),
                pl.BlockSpec((1, 1, _B), lambda h, t, te, tv: (t, 0, 0)),
                pl.BlockSpec((1, _B, 1), lambda h, t, te, tv: (t, 0, 0)),
            ],
            out_specs=pl.BlockSpec((T, D), lambda h, t, te, tv: (0, 0)),
            scratch_shapes=[
                pltpu.VMEM((P_pad, D), jnp.float32),
                pltpu.VMEM((D, _HB), jnp.bfloat16),
                pltpu.VMEM((_HB, D), jnp.bfloat16),
            ],
        ),
        out_shape=jax.ShapeDtypeStruct((T, D), jnp.float32),
    )(tile_expert, tile_valid, xs, W1, b1r, W2, b2r, stok_r, sw_c)

    return out, aux[0, 0]


# speedup vs baseline: 1.1851x; 1.1772x over previous
import jax
import jax.numpy as jnp
from jax import lax
from jax.experimental import pallas as pl
from jax.experimental.pallas import tpu as pltpu

_E = 8
_K = 2
_Z_LOSS_COEF = 0.01
_AUX_LOSS_COEF = 0.01
_B = 256
_HB = 1024


def _router_kernel(x_ref, wr_ref, idx_ref, w_ref, aux_ref, xbf_ref):
    x = x_ref[...]
    logits = jnp.dot(x, wr_ref[...], preferred_element_type=jnp.float32)
    mx = jnp.max(logits, axis=-1, keepdims=True)
    ex = jnp.exp(logits - mx)
    den = jnp.sum(ex, axis=-1, keepdims=True)
    probs = ex / den

    T, E = logits.shape
    iota = lax.broadcasted_iota(jnp.int32, (T, E), 1)
    m1 = jnp.max(probs, axis=-1, keepdims=True)
    idx1 = jnp.min(jnp.where(probs == m1, iota, E), axis=-1, keepdims=True)
    sel1 = iota == idx1
    pm = jnp.where(sel1, -1.0, probs)
    m2 = jnp.max(pm, axis=-1, keepdims=True)
    idx2 = jnp.min(jnp.where(pm == m2, iota, E), axis=-1, keepdims=True)
    sel2 = iota == idx2
    s = m1 + m2
    idx_ref[...] = jnp.concatenate([idx1, idx2], axis=1)
    w_ref[...] = jnp.concatenate([m1 / s, m2 / s], axis=1)

    usage = jnp.mean(probs, axis=0, keepdims=True)
    selection = (
        jnp.mean(sel1.astype(jnp.float32) + sel2.astype(jnp.float32), axis=0,
                 keepdims=True) / _K)
    lb = E * jnp.sum(usage * selection)
    lse = jnp.log(den) + mx
    z = jnp.mean(lse * lse)
    aux_ref[...] = jnp.reshape(_AUX_LOSS_COEF * lb + _Z_LOSS_COEF * z, (1, 1))

    xbf_ref[...] = x.astype(jnp.bfloat16)


def _gather_kernel(tv_ref, xbf_ref, stok_ref, xs_ref):
    t = pl.program_id(0)
    T = xbf_ref.shape[0]
    valid = tv_ref[t] > 0

    @pl.when(valid)
    def _gather():
        stok = stok_ref[0]
        iota = lax.broadcasted_iota(jnp.int32, (_B, T), 1)
        oh = (iota == stok).astype(jnp.bfloat16)
        xs_ref[...] = jnp.dot(
            oh, xbf_ref[...],
            preferred_element_type=jnp.float32).astype(jnp.bfloat16)

    @pl.when(jnp.logical_not(valid))
    def _zero():
        xs_ref[...] = jnp.zeros_like(xs_ref)


def _mlp_kernel(te_ref, tv_ref, xs_ref, w1_ref, b1_ref, w2_ref, b2_ref,
                stokr_ref, sw_ref, out_ref, acc_ref, w1bf_ref, w2bf_ref):
    h = pl.program_id(0)
    t = pl.program_id(1)
    nh = pl.num_programs(0)
    T = out_ref.shape[0]
    valid = tv_ref[t] > 0
    new_w = (t == 0) | (te_ref[t] != te_ref[jnp.maximum(t - 1, 0)])

    @pl.when((h == 0) & (t == 0))
    def _init():
        out_ref[...] = jnp.zeros_like(out_ref)

    @pl.when(valid & new_w)
    def _cast():
        w1bf_ref[...] = w1_ref[0].astype(jnp.bfloat16)
        w2bf_ref[...] = w2_ref[0].astype(jnp.bfloat16)

    @pl.when(valid)
    def _mlp():
        xs = xs_ref[...]
        hpre = jnp.dot(xs, w1bf_ref[...],
                       preferred_element_type=jnp.float32) + b1_ref[0]
        hact = 0.5 * hpre * (1.0 + lax.erf(hpre * 0.7071067811865476))
        hblk = (hact * sw_ref[0]).astype(jnp.bfloat16)
        part = jnp.dot(hblk, w2bf_ref[...], preferred_element_type=jnp.float32)

        @pl.when(h == 0)
        def _set():
            acc_ref[pl.ds(t * _B, _B), :] = part

        @pl.when(h > 0)
        def _add():
            acc_ref[pl.ds(t * _B, _B), :] += part

    @pl.when(valid & (h == nh - 1))
    def _scatter():
        y = acc_ref[pl.ds(t * _B, _B), :] + sw_ref[0] * b2_ref[0]
        stokr = stokr_ref[0]
        iota = lax.broadcasted_iota(jnp.int32, (T, _B), 0)
        ohT = (iota == stokr).astype(jnp.bfloat16)
        out_ref[...] += jnp.dot(ohT, y.astype(jnp.bfloat16),
                                preferred_element_type=jnp.float32)


def kernel(x, Wr, W1, b1, W2, b2):
    T, D = x.shape
    E = Wr.shape[1]
    H = W1.shape[2]
    P = T * _K
    ntiles = P // _B + _E
    P_pad = ntiles * _B

    idx, w, aux, xbf = pl.pallas_call(
        _router_kernel,
        out_shape=[
            jax.ShapeDtypeStruct((T, _K), jnp.int32),
            jax.ShapeDtypeStruct((T, _K), jnp.float32),
            jax.ShapeDtypeStruct((1, 1), jnp.float32),
            jax.ShapeDtypeStruct((T, D), jnp.bfloat16),
        ],
    )(x, Wr)

    idxf = idx.reshape(P)
    wf = w.reshape(P)
    oh = (idxf[:, None] == jnp.arange(E, dtype=jnp.int32)[None, :])
    oh = oh.astype(jnp.int32)
    csum = jnp.cumsum(oh, axis=0)
    ranks = jnp.sum(csum * oh, axis=1) - 1
    counts = csum[-1]
    ptiles = (counts + _B - 1) // _B
    group_start = (jnp.concatenate(
        [jnp.zeros(1, jnp.int32), jnp.cumsum(ptiles)])[:E] * _B)
    pos = group_start[idxf] + ranks
    slot_token = jnp.zeros((P_pad,), jnp.int32).at[pos].set(
        jnp.arange(P, dtype=jnp.int32) // _K)
    slot_w = jnp.zeros((P_pad,), jnp.float32).at[pos].set(wf)
    tile_ids = jnp.arange(ntiles, dtype=jnp.int32)
    tile_bound = jnp.cumsum(ptiles)
    tile_expert = jnp.minimum(
        jnp.searchsorted(tile_bound, tile_ids, side="right").astype(jnp.int32),
        E - 1)
    tile_valid = (tile_ids < tile_bound[-1]).astype(jnp.int32)

    stok_c = slot_token.reshape(ntiles, _B, 1)
    stok_r = slot_token.reshape(ntiles, 1, _B)
    sw_c = slot_w.reshape(ntiles, _B, 1)
    b1r = b1.reshape(E, 1, H)
    b2r = b2.reshape(E, 1, D)
    nh = H // _HB

    xs = pl.pallas_call(
        _gather_kernel,
        grid_spec=pltpu.PrefetchScalarGridSpec(
            num_scalar_prefetch=1,
            grid=(ntiles,),
            in_specs=[
                pl.BlockSpec((T, D), lambda t, tv: (0, 0)),
                pl.BlockSpec((1, _B, 1), lambda t, tv: (t, 0, 0)),
            ],
            out_specs=pl.BlockSpec((_B, D), lambda t, tv: (t, 0)),
        ),
        out_shape=jax.ShapeDtypeStruct((P_pad, D), jnp.bfloat16),
    )(tile_valid, xbf, stok_c)

    out = pl.pallas_call(
        _mlp_kernel,
        grid_spec=pltpu.PrefetchScalarGridSpec(
            num_scalar_prefetch=2,
            grid=(nh, ntiles),
            in_specs=[
                pl.BlockSpec((_B, D), lambda h, t, te, tv: (t, 0)),
                pl.BlockSpec((1, D, _HB), lambda h, t, te, tv: (te[t], 0, h)),
                pl.BlockSpec((1, 1, _HB), lambda h, t, te, tv: (te[t], 0, h)),
                pl.BlockSpec((1, _HB, D), lambda h, t, te, tv: (te[t], h, 0)),
                pl.BlockSpec((1, 1, D), lambda h, t, te, tv: (te[t], 0, 0)),
                pl.BlockSpec((1, 1, _B), lambda h, t, te, tv: (t, 0, 0)),
                pl.BlockSpec((1, _B, 1), lambda h, t, te, tv: (t, 0, 0)),
            ],
            out_specs=pl.BlockSpec((T, D), lambda h, t, te, tv: (0, 0)),
            scratch_shapes=[
                pltpu.VMEM((P_pad, D), jnp.float32),
                pltpu.VMEM((D, _HB), jnp.bfloat16),
                pltpu.VMEM((_HB, D), jnp.bfloat16),
            ],
        ),
        out_shape=jax.ShapeDtypeStruct((T, D), jnp.float32),
    )(tile_expert, tile_valid, xs, W1, b1r, W2, b2r, stok_r, sw_c)

    return out, aux[0, 0]


# speedup vs baseline: 1.2842x; 1.0836x over previous
import jax
import jax.numpy as jnp
from jax import lax
from jax.experimental import pallas as pl
from jax.experimental.pallas import tpu as pltpu

_E = 8
_K = 2
_Z_LOSS_COEF = 0.01
_AUX_LOSS_COEF = 0.01
_B = 512
_HB = 1024


def _router_kernel(x_ref, wr_ref, idx_ref, w_ref, aux_ref, xbf_ref):
    x = x_ref[...]
    logits = jnp.dot(x, wr_ref[...], preferred_element_type=jnp.float32)
    mx = jnp.max(logits, axis=-1, keepdims=True)
    ex = jnp.exp(logits - mx)
    den = jnp.sum(ex, axis=-1, keepdims=True)
    probs = ex / den

    T, E = logits.shape
    iota = lax.broadcasted_iota(jnp.int32, (T, E), 1)
    m1 = jnp.max(probs, axis=-1, keepdims=True)
    idx1 = jnp.min(jnp.where(probs == m1, iota, E), axis=-1, keepdims=True)
    sel1 = iota == idx1
    pm = jnp.where(sel1, -1.0, probs)
    m2 = jnp.max(pm, axis=-1, keepdims=True)
    idx2 = jnp.min(jnp.where(pm == m2, iota, E), axis=-1, keepdims=True)
    sel2 = iota == idx2
    s = m1 + m2
    idx_ref[...] = jnp.concatenate([idx1, idx2], axis=1)
    w_ref[...] = jnp.concatenate([m1 / s, m2 / s], axis=1)

    usage = jnp.mean(probs, axis=0, keepdims=True)
    selection = (
        jnp.mean(sel1.astype(jnp.float32) + sel2.astype(jnp.float32), axis=0,
                 keepdims=True) / _K)
    lb = E * jnp.sum(usage * selection)
    lse = jnp.log(den) + mx
    z = jnp.mean(lse * lse)
    aux_ref[...] = jnp.reshape(_AUX_LOSS_COEF * lb + _Z_LOSS_COEF * z, (1, 1))

    xbf_ref[...] = x.astype(jnp.bfloat16)


def _gather_kernel(tv_ref, xbf_ref, stok_ref, xs_ref):
    t = pl.program_id(0)
    T = xbf_ref.shape[0]
    valid = tv_ref[t] > 0

    @pl.when(valid)
    def _gather():
        stok = stok_ref[0]
        iota = lax.broadcasted_iota(jnp.int32, (_B, T), 1)
        oh = (iota == stok).astype(jnp.bfloat16)
        xs_ref[...] = jnp.dot(
            oh, xbf_ref[...],
            preferred_element_type=jnp.float32).astype(jnp.bfloat16)

    @pl.when(jnp.logical_not(valid))
    def _zero():
        xs_ref[...] = jnp.zeros_like(xs_ref)


def _mlp_kernel(te_ref, tv_ref, xs_ref, w1_ref, b1_ref, w2_ref, b2_ref,
                stokr_ref, sw_ref, out_ref, acc_ref, w1bf_ref, w2bf_ref):
    h = pl.program_id(0)
    t = pl.program_id(1)
    nh = pl.num_programs(0)
    T = out_ref.shape[0]
    valid = tv_ref[t] > 0
    new_w = (t == 0) | (te_ref[t] != te_ref[jnp.maximum(t - 1, 0)])

    @pl.when((h == 0) & (t == 0))
    def _init():
        out_ref[...] = jnp.zeros_like(out_ref)

    @pl.when(valid & new_w)
    def _cast():
        w1bf_ref[...] = w1_ref[0].astype(jnp.bfloat16)
        w2bf_ref[...] = w2_ref[0].astype(jnp.bfloat16)

    @pl.when(valid)
    def _mlp():
        xs = xs_ref[...]
        hpre = jnp.dot(xs, w1bf_ref[...],
                       preferred_element_type=jnp.float32) + b1_ref[0]
        hact = 0.5 * hpre * (1.0 + lax.erf(hpre * 0.7071067811865476))
        hblk = (hact * sw_ref[0]).astype(jnp.bfloat16)
        part = jnp.dot(hblk, w2bf_ref[...], preferred_element_type=jnp.float32)

        @pl.when(h == 0)
        def _set():
            acc_ref[pl.ds(t * _B, _B), :] = part.astype(jnp.bfloat16)

        @pl.when(h > 0)
        def _add():
            acc_ref[pl.ds(t * _B, _B), :] = (
                acc_ref[pl.ds(t * _B, _B), :].astype(jnp.float32) + part
            ).astype(jnp.bfloat16)

    @pl.when(valid & (h == nh - 1))
    def _scatter():
        y = (acc_ref[pl.ds(t * _B, _B), :].astype(jnp.float32)
             + sw_ref[0] * b2_ref[0])
        stokr = stokr_ref[0]
        iota = lax.broadcasted_iota(jnp.int32, (T, _B), 0)
        ohT = (iota == stokr).astype(jnp.bfloat16)
        out_ref[...] += jnp.dot(ohT, y.astype(jnp.bfloat16),
                                preferred_element_type=jnp.float32)


def kernel(x, Wr, W1, b1, W2, b2):
    T, D = x.shape
    E = Wr.shape[1]
    H = W1.shape[2]
    P = T * _K
    ntiles = P // _B + _E - 1
    P_pad = ntiles * _B

    idx, w, aux, xbf = pl.pallas_call(
        _router_kernel,
        out_shape=[
            jax.ShapeDtypeStruct((T, _K), jnp.int32),
            jax.ShapeDtypeStruct((T, _K), jnp.float32),
            jax.ShapeDtypeStruct((1, 1), jnp.float32),
            jax.ShapeDtypeStruct((T, D), jnp.bfloat16),
        ],
    )(x, Wr)

    idxf = idx.reshape(P)
    wf = w.reshape(P)
    oh = (idxf[:, None] == jnp.arange(E, dtype=jnp.int32)[None, :])
    oh = oh.astype(jnp.int32)
    csum = jnp.cumsum(oh, axis=0)
    ranks = jnp.sum(csum * oh, axis=1) - 1
    counts = csum[-1]
    ptiles = (counts + _B - 1) // _B
    group_start = (jnp.concatenate(
        [jnp.zeros(1, jnp.int32), jnp.cumsum(ptiles)])[:E] * _B)
    pos = group_start[idxf] + ranks
    slot_token = jnp.zeros((P_pad,), jnp.int32).at[pos].set(
        jnp.arange(P, dtype=jnp.int32) // _K)
    slot_w = jnp.zeros((P_pad,), jnp.float32).at[pos].set(wf)
    tile_ids = jnp.arange(ntiles, dtype=jnp.int32)
    tile_bound = jnp.cumsum(ptiles)
    tile_expert = jnp.minimum(
        jnp.searchsorted(tile_bound, tile_ids, side="right").astype(jnp.int32),
        E - 1)
    tile_valid = (tile_ids < tile_bound[-1]).astype(jnp.int32)

    stok_c = slot_token.reshape(ntiles, _B, 1)
    stok_r = slot_token.reshape(ntiles, 1, _B)
    sw_c = slot_w.reshape(ntiles, _B, 1)
    b1r = b1.reshape(E, 1, H)
    b2r = b2.reshape(E, 1, D)
    nh = H // _HB

    xs = pl.pallas_call(
        _gather_kernel,
        grid_spec=pltpu.PrefetchScalarGridSpec(
            num_scalar_prefetch=1,
            grid=(ntiles,),
            in_specs=[
                pl.BlockSpec((T, D), lambda t, tv: (0, 0)),
                pl.BlockSpec((1, _B, 1), lambda t, tv: (t, 0, 0)),
            ],
            out_specs=pl.BlockSpec((_B, D), lambda t, tv: (t, 0)),
        ),
        out_shape=jax.ShapeDtypeStruct((P_pad, D), jnp.bfloat16),
    )(tile_valid, xbf, stok_c)

    out = pl.pallas_call(
        _mlp_kernel,
        grid_spec=pltpu.PrefetchScalarGridSpec(
            num_scalar_prefetch=2,
            grid=(nh, ntiles),
            in_specs=[
                pl.BlockSpec((_B, D), lambda h, t, te, tv: (t, 0)),
                pl.BlockSpec((1, D, _HB), lambda h, t, te, tv: (te[t], 0, h)),
                pl.BlockSpec((1, 1, _HB), lambda h, t, te, tv: (te[t], 0, h)),
                pl.BlockSpec((1, _HB, D), lambda h, t, te, tv: (te[t], h, 0)),
                pl.BlockSpec((1, 1, D), lambda h, t, te, tv: (te[t], 0, 0)),
                pl.BlockSpec((1, 1, _B), lambda h, t, te, tv: (t, 0, 0)),
                pl.BlockSpec((1, _B, 1), lambda h, t, te, tv: (t, 0, 0)),
            ],
            out_specs=pl.BlockSpec((T, D), lambda h, t, te, tv: (0, 0)),
            scratch_shapes=[
                pltpu.VMEM((P_pad, D), jnp.bfloat16),
                pltpu.VMEM((D, _HB), jnp.bfloat16),
                pltpu.VMEM((_HB, D), jnp.bfloat16),
            ],
        ),
        out_shape=jax.ShapeDtypeStruct((T, D), jnp.float32),
    )(tile_expert, tile_valid, xs, W1, b1r, W2, b2r, stok_r, sw_c)

    return out, aux[0, 0]


# speedup vs baseline: 1.4606x; 1.1374x over previous
import jax
import jax.numpy as jnp
from jax import lax
from jax.experimental import pallas as pl
from jax.experimental.pallas import tpu as pltpu

_E = 8
_K = 2
_Z_LOSS_COEF = 0.01
_AUX_LOSS_COEF = 0.01
_B = 576
_HB = 1024


def _router_kernel(x_ref, wr_ref, idx_ref, w_ref, aux_ref, xbf_ref):
    x = x_ref[...]
    logits = jnp.dot(x, wr_ref[...], preferred_element_type=jnp.float32)
    mx = jnp.max(logits, axis=-1, keepdims=True)
    ex = jnp.exp(logits - mx)
    den = jnp.sum(ex, axis=-1, keepdims=True)
    probs = ex / den

    T, E = logits.shape
    iota = lax.broadcasted_iota(jnp.int32, (T, E), 1)
    m1 = jnp.max(probs, axis=-1, keepdims=True)
    idx1 = jnp.min(jnp.where(probs == m1, iota, E), axis=-1, keepdims=True)
    sel1 = iota == idx1
    pm = jnp.where(sel1, -1.0, probs)
    m2 = jnp.max(pm, axis=-1, keepdims=True)
    idx2 = jnp.min(jnp.where(pm == m2, iota, E), axis=-1, keepdims=True)
    sel2 = iota == idx2
    s = m1 + m2
    idx_ref[...] = jnp.concatenate([idx1, idx2], axis=1)
    w_ref[...] = jnp.concatenate([m1 / s, m2 / s], axis=1)

    usage = jnp.mean(probs, axis=0, keepdims=True)
    selection = (
        jnp.mean(sel1.astype(jnp.float32) + sel2.astype(jnp.float32), axis=0,
                 keepdims=True) / _K)
    lb = E * jnp.sum(usage * selection)
    lse = jnp.log(den) + mx
    z = jnp.mean(lse * lse)
    aux_ref[...] = jnp.reshape(_AUX_LOSS_COEF * lb + _Z_LOSS_COEF * z, (1, 1))

    xbf_ref[...] = x.astype(jnp.bfloat16)


def _gather_kernel(tv_ref, xbf_ref, stok_ref, xs_ref):
    t = pl.program_id(0)
    T = xbf_ref.shape[0]
    valid = tv_ref[t] > 0

    @pl.when(valid)
    def _gather():
        stok = stok_ref[0]
        iota = lax.broadcasted_iota(jnp.int32, (_B, T), 1)
        oh = (iota == stok).astype(jnp.bfloat16)
        xs_ref[...] = jnp.dot(
            oh, xbf_ref[...],
            preferred_element_type=jnp.float32).astype(jnp.bfloat16)

    @pl.when(jnp.logical_not(valid))
    def _zero():
        xs_ref[...] = jnp.zeros_like(xs_ref)


def _mlp_kernel(te_ref, tv_ref, xs_ref, w1_ref, b1_ref, w2_ref, b2_ref,
                stokr_ref, sw_ref, out_ref, acc_ref, w1bf_ref, w2bf_ref):
    h = pl.program_id(0)
    t = pl.program_id(1)
    nh = pl.num_programs(0)
    T = out_ref.shape[0]
    valid = tv_ref[t] > 0
    new_w = (t == 0) | (te_ref[t] != te_ref[jnp.maximum(t - 1, 0)])

    @pl.when((h == 0) & (t == 0))
    def _init():
        out_ref[...] = jnp.zeros_like(out_ref)

    @pl.when(valid & new_w)
    def _cast():
        w1bf_ref[...] = w1_ref[0].astype(jnp.bfloat16)
        w2bf_ref[...] = w2_ref[0].astype(jnp.bfloat16)

    @pl.when(valid)
    def _mlp():
        xs = xs_ref[...]
        hpre = jnp.dot(xs, w1bf_ref[...],
                       preferred_element_type=jnp.float32) + b1_ref[0]
        hact = 0.5 * hpre * (1.0 + lax.erf(hpre * 0.7071067811865476))
        hblk = (hact * sw_ref[0]).astype(jnp.bfloat16)
        part = jnp.dot(hblk, w2bf_ref[...], preferred_element_type=jnp.float32)

        @pl.when(h == 0)
        def _set():
            acc_ref[pl.ds(t * _B, _B), :] = part.astype(jnp.bfloat16)

        @pl.when(h > 0)
        def _add():
            acc_ref[pl.ds(t * _B, _B), :] = (
                acc_ref[pl.ds(t * _B, _B), :].astype(jnp.float32) + part
            ).astype(jnp.bfloat16)

    @pl.when(valid & (h == nh - 1))
    def _scatter():
        y = (acc_ref[pl.ds(t * _B, _B), :].astype(jnp.float32)
             + sw_ref[0] * b2_ref[0])
        stokr = stokr_ref[0]
        iota = lax.broadcasted_iota(jnp.int32, (T, _B), 0)
        ohT = (iota == stokr).astype(jnp.bfloat16)
        out_ref[...] += jnp.dot(ohT, y.astype(jnp.bfloat16),
                                preferred_element_type=jnp.float32)


def kernel(x, Wr, W1, b1, W2, b2):
    T, D = x.shape
    E = Wr.shape[1]
    H = W1.shape[2]
    P = T * _K
    ntiles = -(-P // _B) + _E - 1
    P_pad = ntiles * _B

    idx, w, aux, xbf = pl.pallas_call(
        _router_kernel,
        out_shape=[
            jax.ShapeDtypeStruct((T, _K), jnp.int32),
            jax.ShapeDtypeStruct((T, _K), jnp.float32),
            jax.ShapeDtypeStruct((1, 1), jnp.float32),
            jax.ShapeDtypeStruct((T, D), jnp.bfloat16),
        ],
    )(x, Wr)

    idxf = idx.reshape(P)
    wf = w.reshape(P)
    oh = (idxf[:, None] == jnp.arange(E, dtype=jnp.int32)[None, :])
    oh = oh.astype(jnp.int32)
    csum = jnp.cumsum(oh, axis=0)
    ranks = jnp.sum(csum * oh, axis=1) - 1
    counts = csum[-1]
    ptiles = (counts + _B - 1) // _B
    group_start = (jnp.concatenate(
        [jnp.zeros(1, jnp.int32), jnp.cumsum(ptiles)])[:E] * _B)
    pos = group_start[idxf] + ranks
    slot_token = jnp.zeros((P_pad,), jnp.int32).at[pos].set(
        jnp.arange(P, dtype=jnp.int32) // _K)
    slot_w = jnp.zeros((P_pad,), jnp.float32).at[pos].set(wf)
    tile_ids = jnp.arange(ntiles, dtype=jnp.int32)
    tile_bound = jnp.cumsum(ptiles)
    tile_expert = jnp.minimum(
        jnp.searchsorted(tile_bound, tile_ids, side="right").astype(jnp.int32),
        E - 1)
    tile_valid = (tile_ids < tile_bound[-1]).astype(jnp.int32)

    stok_c = slot_token.reshape(ntiles, _B, 1)
    stok_r = slot_token.reshape(ntiles, 1, _B)
    sw_c = slot_w.reshape(ntiles, _B, 1)
    b1r = b1.reshape(E, 1, H)
    b2r = b2.reshape(E, 1, D)
    nh = H // _HB

    xs = pl.pallas_call(
        _gather_kernel,
        grid_spec=pltpu.PrefetchScalarGridSpec(
            num_scalar_prefetch=1,
            grid=(ntiles,),
            in_specs=[
                pl.BlockSpec((T, D), lambda t, tv: (0, 0)),
                pl.BlockSpec((1, _B, 1), lambda t, tv: (t, 0, 0)),
            ],
            out_specs=pl.BlockSpec((_B, D), lambda t, tv: (t, 0)),
        ),
        out_shape=jax.ShapeDtypeStruct((P_pad, D), jnp.bfloat16),
    )(tile_valid, xbf, stok_c)

    out = pl.pallas_call(
        _mlp_kernel,
        grid_spec=pltpu.PrefetchScalarGridSpec(
            num_scalar_prefetch=2,
            grid=(nh, ntiles),
            in_specs=[
                pl.BlockSpec((_B, D), lambda h, t, te, tv: (t, 0)),
                pl.BlockSpec((1, D, _HB), lambda h, t, te, tv: (te[t], 0, h)),
                pl.BlockSpec((1, 1, _HB), lambda h, t, te, tv: (te[t], 0, h)),
                pl.BlockSpec((1, _HB, D), lambda h, t, te, tv: (te[t], h, 0)),
                pl.BlockSpec((1, 1, D), lambda h, t, te, tv: (te[t], 0, 0)),
                pl.BlockSpec((1, 1, _B), lambda h, t, te, tv: (t, 0, 0)),
                pl.BlockSpec((1, _B, 1), lambda h, t, te, tv: (t, 0, 0)),
            ],
            out_specs=pl.BlockSpec((T, D), lambda h, t, te, tv: (0, 0)),
            scratch_shapes=[
                pltpu.VMEM((P_pad, D), jnp.bfloat16),
                pltpu.VMEM((D, _HB), jnp.bfloat16),
                pltpu.VMEM((_HB, D), jnp.bfloat16),
            ],
        ),
        out_shape=jax.ShapeDtypeStruct((T, D), jnp.float32),
    )(tile_expert, tile_valid, xs, W1, b1r, W2, b2r, stok_r, sw_c)

    return out, aux[0, 0]
